# baseline (speedup 1.0000x reference)
import numpy as np
import jax
import jax.numpy as jnp
from jax.experimental import pallas as pl
from jax.experimental.pallas import tpu as pltpu

_B, _H, _W, _CIN = 8, 224, 224, 96
_PH, _PW = 16, 16
_NPW = 14
_KOUT = 256
_LANES = _PW * _CIN
_ROWS = _B * _H * _NPW
_SPLIT = 7
_GRID = _B * _SPLIT
_RB = _ROWS // _GRID
_CB = 18816 // _GRID

_M2 = (np.arange(_LANES)[:, None] // _CIN == np.arange(128)[None, :]).astype(np.float32)
_P2 = ((np.arange(_RB)[None, :] // _NPW) % _PH == np.arange(_PH)[:, None]).astype(np.float32)


def _reduce_kernel(x_ref, m_ref, p_ref, c_ref, o_ref, s_ref, w_ref):
    i = pl.program_id(0)
    b = i // _SPLIT
    j = i % _SPLIT

    @pl.when(i == 0)
    def _init():
        w_ref[...] = jnp.zeros_like(w_ref)

    t = jnp.dot(x_ref[...].astype(jnp.bfloat16), m_ref[...],
                preferred_element_type=jnp.float32)
    sb = jnp.dot(p_ref[...], t.astype(jnp.bfloat16),
                 preferred_element_type=jnp.float32)

    @pl.when(j == 0)
    def _snew():
        s_ref[pl.ds(_PH * b, _PH), :] = sb

    @pl.when(j > 0)
    def _sacc():
        s_ref[pl.ds(_PH * b, _PH), :] += sb

    w_ref[...] += c_ref[...].sum(axis=0, keepdims=True)

    @pl.when(i == _GRID - 1)
    def _fin():
        s = s_ref[...][:, 0:_PW]
        o_ref[...] = s.reshape(_B * _PH, _PW, 1) * w_ref[...].reshape(1, 1, _KOUT)


def kernel(x, C):
    x3 = x.reshape(_ROWS, _LANES)
    c2 = C.reshape(18816, _KOUT)
    m2 = jnp.asarray(_M2, dtype=jnp.bfloat16)
    p2 = jnp.asarray(_P2, dtype=jnp.bfloat16)
    out3 = pl.pallas_call(
        _reduce_kernel,
        grid=(_GRID,),
        in_specs=[
            pl.BlockSpec((_RB, _LANES), lambda i: (i, 0)),
            pl.BlockSpec((_LANES, 128), lambda i: (0, 0)),
            pl.BlockSpec((_PH, _RB), lambda i: (0, 0)),
            pl.BlockSpec((_CB, _KOUT), lambda i: (i, 0)),
        ],
        out_specs=pl.BlockSpec((_B * _PH, _PW, _KOUT), lambda i: (0, 0, 0)),
        out_shape=jax.ShapeDtypeStruct((_B * _PH, _PW, _KOUT), jnp.float32),
        scratch_shapes=[
            pltpu.VMEM((_B * _PH, 128), jnp.float32),
            pltpu.VMEM((1, _KOUT), jnp.float32),
        ],
    )(x3, m2, p2, c2)
    out = out3.reshape(_B, _PH, _PW, _KOUT).transpose(0, 3, 1, 2)
    return out.reshape(_B, _KOUT, 1, _PH, _PW)


# speedup vs baseline: 3.6209x; 3.6209x over previous
import jax
import jax.numpy as jnp
from jax.experimental import pallas as pl
from jax.experimental.pallas import tpu as pltpu

_B, _H, _W, _CIN = 8, 224, 224, 96
_PH, _PW = 16, 16
_NPH, _NPW = 14, 14
_NP = _NPH * _NPW
_KOUT = 256
_M = _B * _PH * _PW


def _reduce_kernel(x_ref, c_ref, o_ref, s_ref, w_ref):
    i = pl.program_id(0)

    @pl.when(i == 0)
    def _init():
        s_ref[...] = jnp.zeros_like(s_ref)
        w_ref[...] = jnp.zeros_like(w_ref)

    acc = x_ref[:, :, 0:_PW, :]
    for j in range(1, _NPW):
        acc = acc + x_ref[:, :, _PW * j:_PW * (j + 1), :]
    s_ref[...] += acc.reshape(_M, _CIN).sum(axis=1, keepdims=True)
    w_ref[...] += c_ref[...].reshape(_NPW * _CIN, _KOUT).sum(axis=0, keepdims=True)

    @pl.when(i == _NPH - 1)
    def _fin():
        o_ref[...] = s_ref[...] * w_ref[...]


def kernel(x, C):
    c3 = C.reshape(_NP, _CIN, _KOUT)
    out2 = pl.pallas_call(
        _reduce_kernel,
        grid=(_NPH,),
        in_specs=[
            pl.BlockSpec((_B, _PH, _W, _CIN), lambda i: (0, i, 0, 0)),
            pl.BlockSpec((_NPW, _CIN, _KOUT), lambda i: (i, 0, 0)),
        ],
        out_specs=pl.BlockSpec((_M, _KOUT), lambda i: (0, 0)),
        out_shape=jax.ShapeDtypeStruct((_M, _KOUT), jnp.float32),
        scratch_shapes=[
            pltpu.VMEM((_M, 1), jnp.float32),
            pltpu.VMEM((1, _KOUT), jnp.float32),
        ],
    )(x, c3)
    out = out2.reshape(_B, _PH, _PW, _KOUT).transpose(0, 3, 1, 2)
    return out.reshape(_B, _KOUT, 1, _PH, _PW)


# speedup vs baseline: 3.6240x; 1.0008x over previous
import jax
import jax.numpy as jnp
from jax.experimental import pallas as pl
from jax.experimental.pallas import tpu as pltpu

_B, _H, _W, _CIN = 8, 224, 224, 96
_PH, _PW = 16, 16
_NPH, _NPW = 14, 14
_NP = _NPH * _NPW
_KOUT = 256
_M = _B * _PH * _PW
_MB = _PH * _PW


def _reduce_kernel(*refs):
    x_refs = refs[:_B]
    c_ref, o_ref, s_ref, w_ref = refs[_B:]
    i = pl.program_id(0)

    @pl.when(i == 0)
    def _init():
        s_ref[...] = jnp.zeros_like(s_ref)
        w_ref[...] = jnp.zeros_like(w_ref)

    for b in range(_B):
        xb = x_refs[b]
        acc = xb[:, :, 0:_PW, :]
        for j in range(1, _NPW):
            acc = acc + xb[:, :, _PW * j:_PW * (j + 1), :]
        s_ref[pl.ds(_MB * b, _MB), :] += acc.reshape(_MB, _CIN).sum(
            axis=1, keepdims=True)
    w_ref[...] += c_ref[...].reshape(_NPW * _CIN, _KOUT).sum(axis=0, keepdims=True)

    @pl.when(i == _NPH - 1)
    def _fin():
        o_ref[...] = s_ref[...] * w_ref[...]


def kernel(x, C):
    c3 = C.reshape(_NP, _CIN, _KOUT)

    def _xspec(b):
        return pl.BlockSpec((1, _PH, _W, _CIN), lambda i, b=b: (b, i, 0, 0))

    out2 = pl.pallas_call(
        _reduce_kernel,
        grid=(_NPH,),
        in_specs=[_xspec(b) for b in range(_B)] + [
            pl.BlockSpec((_NPW, _CIN, _KOUT), lambda i: (i, 0, 0)),
        ],
        out_specs=pl.BlockSpec((_M, _KOUT), lambda i: (0, 0)),
        out_shape=jax.ShapeDtypeStruct((_M, _KOUT), jnp.float32),
        scratch_shapes=[
            pltpu.VMEM((_M, 1), jnp.float32),
            pltpu.VMEM((1, _KOUT), jnp.float32),
        ],
    )(*([x] * _B + [c3]))
    out = out2.reshape(_B, _PH, _PW, _KOUT).transpose(0, 3, 1, 2)
    return out.reshape(_B, _KOUT, 1, _PH, _PW)
